# baseline (speedup 1.0000x reference)
import functools

import jax
import jax.numpy as jnp
from jax import lax
from jax.experimental import pallas as pl
from jax.experimental.pallas import tpu as pltpu
from jax.experimental.pallas import tpu_sc as plsc

NUM_CLASSES = 1000000
DROPOUT_PROB = 0.1
HIDDEN = 64

_info = plsc.get_sparse_core_info()
_NC, _NS = _info.num_cores, _info.num_subcores
_NW = _NC * _NS

_IDX_MINOR = 128


def _gather_kernel(idx_hbm, table_hbm, out_hbm, idx_v, rows_v, sem):
    wid = lax.axis_index("s") * _NC + lax.axis_index("c")
    n_rows_per_w = idx_v.shape[0]
    base_row = wid * n_rows_per_w
    pltpu.sync_copy(idx_hbm.at[pl.ds(base_row, n_rows_per_w)], idx_v)
    for j in range(n_rows_per_w):
        pltpu.async_copy(
            table_hbm.at[idx_v.at[j]],
            rows_v.at[pl.ds(j * _IDX_MINOR, _IDX_MINOR)],
            sem,
        )
    for j in range(n_rows_per_w):
        pltpu.make_async_copy(
            table_hbm.at[idx_v.at[j]],
            rows_v.at[pl.ds(j * _IDX_MINOR, _IDX_MINOR)],
            sem,
        ).wait()
    pltpu.sync_copy(
        rows_v, out_hbm.at[pl.ds(base_row * _IDX_MINOR, n_rows_per_w * _IDX_MINOR)]
    )


def kernel(labels, train, embedding_table):
    B = labels.shape[0]
    D = embedding_table.shape[1]
    key = jax.random.key(42)
    drop_ids = jax.random.uniform(key, (B,)) < DROPOUT_PROB
    dropped = jnp.where(drop_ids, NUM_CLASSES, labels)
    idx = jnp.where(train != 0, dropped, labels).astype(jnp.int32)

    b_per_w = B // _NW
    n_rows_per_w = b_per_w // _IDX_MINOR
    idx2d = idx.reshape(B // _IDX_MINOR, _IDX_MINOR)

    mesh = plsc.VectorSubcoreMesh(core_axis_name="c", subcore_axis_name="s")
    run = functools.partial(
        pl.kernel,
        _gather_kernel,
        mesh=mesh,
        compiler_params=pltpu.CompilerParams(use_tc_tiling_on_sc=False),
        out_type=jax.ShapeDtypeStruct((B, D), jnp.float32),
        scratch_types=[
            pltpu.VMEM((n_rows_per_w, _IDX_MINOR), jnp.int32),
            pltpu.VMEM((b_per_w, D), jnp.float32),
            pltpu.SemaphoreType.DMA,
        ],
    )()
    return run(idx2d, embedding_table)


# speedup vs baseline: 1.7211x; 1.7211x over previous
import functools

import jax
import jax.numpy as jnp
from jax import lax
from jax.experimental import pallas as pl
from jax.experimental.pallas import tpu as pltpu
from jax.experimental.pallas import tpu_sc as plsc

NUM_CLASSES = 1000000
DROPOUT_PROB = 0.1

_info = plsc.get_sparse_core_info()
_NC, _NS = _info.num_cores, _info.num_subcores
_NW = _NC * _NS


def _gather_kernel(idx_hbm, table_hbm, out_hbm, idx_v, rows_v, sem):
    wid = lax.axis_index("s") * _NC + lax.axis_index("c")
    b_per_w = rows_v.shape[0]
    base = wid * b_per_w
    pltpu.sync_copy(idx_hbm.at[pl.ds(base, b_per_w)], idx_v)

    def issue(g, carry):
        vec = idx_v[pl.ds(g * 16, 16)]
        for j in range(16):
            r = vec[j]
            pltpu.async_copy(
                table_hbm.at[pl.ds(r, 1)],
                rows_v.at[pl.ds(g * 16 + j, 1)],
                sem,
            )
        return carry

    lax.fori_loop(0, b_per_w // 16, issue, 0)
    pltpu.make_async_copy(table_hbm.at[pl.ds(0, b_per_w)], rows_v, sem).wait()
    pltpu.sync_copy(rows_v, out_hbm.at[pl.ds(base, b_per_w)])


def kernel(labels, train, embedding_table):
    B = labels.shape[0]
    D = embedding_table.shape[1]
    key = jax.random.key(42)
    drop_ids = jax.random.uniform(key, (B,)) < DROPOUT_PROB
    dropped = jnp.where(drop_ids, NUM_CLASSES, labels)
    idx = jnp.where(train != 0, dropped, labels).astype(jnp.int32)

    b_per_w = B // _NW

    mesh = plsc.VectorSubcoreMesh(core_axis_name="c", subcore_axis_name="s")
    run = functools.partial(
        pl.kernel,
        _gather_kernel,
        mesh=mesh,
        out_type=jax.ShapeDtypeStruct((B, D), jnp.float32),
        scratch_types=[
            pltpu.VMEM((b_per_w,), jnp.int32),
            pltpu.VMEM((b_per_w, D), jnp.float32),
            pltpu.SemaphoreType.DMA,
        ],
    )()
    return run(idx, embedding_table)
